# baseline (speedup 1.0000x reference)
import functools

import jax
import jax.numpy as jnp
from jax import lax
from jax.experimental import pallas as pl
from jax.experimental.pallas import tpu as pltpu
from jax.experimental.pallas import tpu_sc as plsc

VOCAB = 100000
DIM = 128
RANK = 256
SCALING = 512.0 / 256.0
N_TOK = 1024 * 20

NC = 2
NS = 16
L = 16
NW = NC * NS
R_PER_W = RANK // NW
TOK_PER_W = N_TOK // NW
CHUNK = 2048
NCHUNK = N_TOK // CHUNK

_mesh = plsc.VectorSubcoreMesh(core_axis_name="c", subcore_axis_name="s")


@functools.partial(
    pl.kernel,
    mesh=_mesh,
    out_type=jax.ShapeDtypeStruct((N_TOK, DIM), jnp.float32),
    scratch_types=[
        pltpu.VMEM((TOK_PER_W,), jnp.int32),
        pltpu.VMEM((TOK_PER_W, DIM), jnp.float32),
        pltpu.SemaphoreType.DMA,
    ],
)
def _base_gather(table_hbm, idx_hbm, out_hbm, idx_v, rows_v, sem):
    wid = lax.axis_index("s") * NC + lax.axis_index("c")
    base = wid * TOK_PER_W
    pltpu.sync_copy(idx_hbm.at[pl.ds(base, TOK_PER_W)], idx_v)
    pltpu.async_copy(table_hbm.at[idx_v], rows_v, sem).wait()
    pltpu.sync_copy(rows_v, out_hbm.at[pl.ds(base, TOK_PER_W)])


@functools.partial(
    pl.kernel,
    mesh=_mesh,
    out_type=jax.ShapeDtypeStruct((RANK, N_TOK), jnp.float32),
    compiler_params=pltpu.CompilerParams(needs_layout_passes=False),
    scratch_types=[
        pltpu.VMEM((VOCAB,), jnp.float32),
        pltpu.VMEM((N_TOK,), jnp.int32),
        pltpu.VMEM((CHUNK,), jnp.float32),
    ],
)
def _z_gather(a_hbm, idx_hbm, z_hbm, row_v, ids_v, chunk_v):
    wid = lax.axis_index("s") * NC + lax.axis_index("c")
    pltpu.sync_copy(idx_hbm, ids_v)

    def row_body(j, carry):
        r = wid * R_PER_W + j
        pltpu.sync_copy(a_hbm.at[r], row_v)

        def chunk_body(ci, carry2):
            def vec_body(k, carry3):
                off = ci * CHUNK + k * L
                idx = ids_v[pl.ds(off, L)]
                vals = plsc.load_gather(row_v, [idx])
                chunk_v[pl.ds(k * L, L)] = vals
                return carry3

            lax.fori_loop(0, CHUNK // L, vec_body, 0)
            pltpu.sync_copy(chunk_v, z_hbm.at[r, pl.ds(ci * CHUNK, CHUNK)])
            return carry2

        lax.fori_loop(0, NCHUNK, chunk_body, 0)
        return carry

    lax.fori_loop(0, R_PER_W, row_body, 0)


TBLK = 2048


def _tc_body(z_ref, bg_ref, b_ref, out_ref):
    acc = lax.dot_general(
        z_ref[...], b_ref[...],
        (((0,), (1,)), ((), ())),
        preferred_element_type=jnp.float32,
        precision=lax.Precision.HIGHEST,
    )
    out_ref[...] = bg_ref[...] + acc * SCALING


def _tc_finish(z, base_g, lora_B):
    return pl.pallas_call(
        _tc_body,
        grid=(N_TOK // TBLK,),
        in_specs=[
            pl.BlockSpec((RANK, TBLK), lambda i: (0, i)),
            pl.BlockSpec((TBLK, DIM), lambda i: (i, 0)),
            pl.BlockSpec((DIM, RANK), lambda i: (0, 0)),
        ],
        out_specs=pl.BlockSpec((TBLK, DIM), lambda i: (i, 0)),
        out_shape=jax.ShapeDtypeStruct((N_TOK, DIM), jnp.float32),
    )(z, base_g, lora_B)


def kernel(input_ids, base_weight, lora_A, lora_B):
    ids = input_ids.reshape(-1).astype(jnp.int32)
    base_g = _base_gather(base_weight, ids)
    z = _z_gather(lora_A, ids)
    out = _tc_finish(z, base_g, lora_B)
    return out.reshape(*input_ids.shape, DIM)


# speedup vs baseline: 1.3912x; 1.3912x over previous
import functools

import jax
import jax.numpy as jnp
from jax import lax
from jax.experimental import pallas as pl
from jax.experimental.pallas import tpu as pltpu
from jax.experimental.pallas import tpu_sc as plsc

VOCAB = 100000
DIM = 128
RANK = 256
SCALING = 512.0 / 256.0
N_TOK = 1024 * 20

NC = 2
NS = 16
L = 16
NW = NC * NS
R_PER_W = RANK // NW
TOK_PER_W = N_TOK // NW
CHUNK = 1024
NCHUNK = N_TOK // CHUNK

_mesh = plsc.VectorSubcoreMesh(core_axis_name="c", subcore_axis_name="s")


@functools.partial(
    pl.kernel,
    mesh=_mesh,
    out_type=jax.ShapeDtypeStruct((N_TOK, DIM), jnp.float32),
    scratch_types=[
        pltpu.VMEM((TOK_PER_W,), jnp.int32),
        pltpu.VMEM((TOK_PER_W, DIM), jnp.float32),
        pltpu.SemaphoreType.DMA,
    ],
)
def _base_gather(table_hbm, idx_hbm, out_hbm, idx_v, rows_v, sem):
    wid = lax.axis_index("s") * NC + lax.axis_index("c")
    base = wid * TOK_PER_W
    pltpu.sync_copy(idx_hbm.at[pl.ds(base, TOK_PER_W)], idx_v)
    pltpu.async_copy(table_hbm.at[idx_v], rows_v, sem).wait()
    pltpu.sync_copy(rows_v, out_hbm.at[pl.ds(base, TOK_PER_W)])


@functools.partial(
    pl.kernel,
    mesh=_mesh,
    out_type=jax.ShapeDtypeStruct((RANK, N_TOK), jnp.float32),
    compiler_params=pltpu.CompilerParams(needs_layout_passes=False),
    scratch_types=[
        pltpu.VMEM((VOCAB,), jnp.float32),
        pltpu.VMEM((N_TOK,), jnp.int32),
        pltpu.VMEM((2, CHUNK), jnp.float32),
        pltpu.SemaphoreType.DMA,
    ],
)
def _z_gather(a_hbm, idx_hbm, z_hbm, row_v, ids_v, chunk_v, sem):
    wid = lax.axis_index("s") * NC + lax.axis_index("c")
    pltpu.sync_copy(idx_hbm, ids_v)

    def row_body(j, carry):
        r = wid * R_PER_W + j
        pltpu.sync_copy(a_hbm.at[r], row_v)

        descs = {}
        for c in range(NCHUNK):
            p = c % 2
            if c >= 2:
                descs[c - 2].wait()

            @plsc.parallel_loop(0, CHUNK // L, unroll=8)
            def fill(i, _c=c, _p=p):
                off = _c * CHUNK + i * L
                idx = ids_v[pl.ds(off, L)]
                chunk_v[_p, pl.ds(i * L, L)] = plsc.load_gather(row_v, [idx])

            descs[c] = pltpu.async_copy(
                chunk_v.at[p], z_hbm.at[r, pl.ds(c * CHUNK, CHUNK)], sem
            )
        descs[NCHUNK - 2].wait()
        descs[NCHUNK - 1].wait()
        return carry

    lax.fori_loop(0, R_PER_W, row_body, 0)


TBLK = 2048


def _tc_body(z_ref, bg_ref, b_ref, out_ref):
    acc = lax.dot_general(
        z_ref[...], b_ref[...],
        (((0,), (1,)), ((), ())),
        preferred_element_type=jnp.float32,
        precision=lax.Precision.HIGHEST,
    )
    out_ref[...] = bg_ref[...] + acc * SCALING


def _tc_finish(z, base_g, lora_B):
    return pl.pallas_call(
        _tc_body,
        grid=(N_TOK // TBLK,),
        in_specs=[
            pl.BlockSpec((RANK, TBLK), lambda i: (0, i)),
            pl.BlockSpec((TBLK, DIM), lambda i: (i, 0)),
            pl.BlockSpec((DIM, RANK), lambda i: (0, 0)),
        ],
        out_specs=pl.BlockSpec((TBLK, DIM), lambda i: (i, 0)),
        out_shape=jax.ShapeDtypeStruct((N_TOK, DIM), jnp.float32),
    )(z, base_g, lora_B)


def kernel(input_ids, base_weight, lora_A, lora_B):
    ids = input_ids.reshape(-1).astype(jnp.int32)
    base_g = _base_gather(base_weight, ids)
    z = _z_gather(lora_A, ids)
    out = _tc_finish(z, base_g, lora_B)
    return out.reshape(*input_ids.shape, DIM)


# speedup vs baseline: 3.4499x; 2.4798x over previous
import functools

import jax
import jax.numpy as jnp
from jax import lax
from jax.experimental import pallas as pl
from jax.experimental.pallas import tpu as pltpu
from jax.experimental.pallas import tpu_sc as plsc

VOCAB = 100000
DIM = 128
RANK = 256
SCALING = 512.0 / 256.0
N_TOK = 1024 * 20

NC = 2
NS = 16
NW = NC * NS
TOK_PER_W = N_TOK // NW
CKT = 128
NCKT = TOK_PER_W // CKT

_mesh = plsc.VectorSubcoreMesh(core_axis_name="c", subcore_axis_name="s")


def _pipelined_gather(table_hbm, idx_v, out_hbm, base, bufs, gsems, ssems):
    g = {}
    s = {}
    g[0] = pltpu.async_copy(
        table_hbm.at[idx_v.at[pl.ds(0, CKT)]], bufs.at[0], gsems[0]
    )
    for c in range(NCKT):
        p = c % 2
        if c + 1 < NCKT:
            if c >= 1:
                s[c - 1].wait()
            g[c + 1] = pltpu.async_copy(
                table_hbm.at[idx_v.at[pl.ds((c + 1) * CKT, CKT)]],
                bufs.at[1 - p],
                gsems[1 - p],
            )
        g[c].wait()
        s[c] = pltpu.async_copy(
            bufs.at[p], out_hbm.at[pl.ds(base + c * CKT, CKT)], ssems[p]
        )
    s[NCKT - 2].wait()
    s[NCKT - 1].wait()


@functools.partial(
    pl.kernel,
    mesh=_mesh,
    out_type=(
        jax.ShapeDtypeStruct((N_TOK, RANK), jnp.float32),
        jax.ShapeDtypeStruct((N_TOK, DIM), jnp.float32),
    ),
    compiler_params=pltpu.CompilerParams(use_tc_tiling_on_sc=True),
    scratch_types=[
        pltpu.VMEM((TOK_PER_W,), jnp.int32),
        pltpu.VMEM((2, CKT, RANK), jnp.float32),
        pltpu.VMEM((2, CKT, DIM), jnp.float32),
        pltpu.SemaphoreType.DMA,
        pltpu.SemaphoreType.DMA,
        pltpu.SemaphoreType.DMA,
        pltpu.SemaphoreType.DMA,
    ],
)
def _sc_gathers(
    at_hbm, bw_hbm, idx_hbm, zt_hbm, bg_hbm,
    idx_v, zbuf, bbuf, sem_a, sem_b, sem_c, sem_d,
):
    wid = lax.axis_index("s") * NC + lax.axis_index("c")
    base = wid * TOK_PER_W
    pltpu.sync_copy(idx_hbm.at[pl.ds(base, TOK_PER_W)], idx_v)
    _pipelined_gather(at_hbm, idx_v, zt_hbm, base, zbuf, (sem_a, sem_b), (sem_c, sem_d))
    _pipelined_gather(bw_hbm, idx_v, bg_hbm, base, bbuf, (sem_a, sem_b), (sem_c, sem_d))


TBLK = 2048


def _tc_body(zt_ref, bg_ref, b_ref, out_ref):
    acc = lax.dot_general(
        zt_ref[...], b_ref[...],
        (((1,), (1,)), ((), ())),
        preferred_element_type=jnp.float32,
        precision=lax.Precision.HIGHEST,
    )
    out_ref[...] = bg_ref[...] + acc * SCALING


def _tc_finish(zt, base_g, lora_B):
    return pl.pallas_call(
        _tc_body,
        grid=(N_TOK // TBLK,),
        in_specs=[
            pl.BlockSpec((TBLK, RANK), lambda i: (i, 0)),
            pl.BlockSpec((TBLK, DIM), lambda i: (i, 0)),
            pl.BlockSpec((DIM, RANK), lambda i: (0, 0)),
        ],
        out_specs=pl.BlockSpec((TBLK, DIM), lambda i: (i, 0)),
        out_shape=jax.ShapeDtypeStruct((N_TOK, DIM), jnp.float32),
    )(zt, base_g, lora_B)


def kernel(input_ids, base_weight, lora_A, lora_B):
    ids = input_ids.reshape(-1).astype(jnp.int32)
    at = lora_A.T
    zt, base_g = _sc_gathers(at, base_weight, ids)
    out = _tc_finish(zt, base_g, lora_B)
    return out.reshape(*input_ids.shape, DIM)


# speedup vs baseline: 4.6981x; 1.3618x over previous
import functools

import jax
import jax.numpy as jnp
from jax import lax
from jax.experimental import pallas as pl
from jax.experimental.pallas import tpu as pltpu
from jax.experimental.pallas import tpu_sc as plsc

VOCAB = 100000
DIM = 128
RANK = 256
SCALING = 512.0 / 256.0
N_TOK = 1024 * 20

NC = 2
NS = 16
NW = NC * NS
TOK_PER_W = N_TOK // NW
CKT = 128
NCKT = TOK_PER_W // CKT

_mesh = plsc.VectorSubcoreMesh(core_axis_name="c", subcore_axis_name="s")


def _pipelined_gather(table_hbm, idx_v, out_hbm, base, bufs, gsems, ssems):
    g = {}
    s = {}
    g[0] = pltpu.async_copy(
        table_hbm.at[idx_v.at[pl.ds(0, CKT)]], bufs.at[0], gsems[0]
    )
    for c in range(NCKT):
        p = c % 2
        if c + 1 < NCKT:
            if c >= 1:
                s[c - 1].wait()
            g[c + 1] = pltpu.async_copy(
                table_hbm.at[idx_v.at[pl.ds((c + 1) * CKT, CKT)]],
                bufs.at[1 - p],
                gsems[1 - p],
            )
        g[c].wait()
        s[c] = pltpu.async_copy(
            bufs.at[p], out_hbm.at[pl.ds(base + c * CKT, CKT)], ssems[p]
        )
    s[NCKT - 2].wait()
    s[NCKT - 1].wait()


@functools.partial(
    pl.kernel,
    mesh=_mesh,
    out_type=(
        jax.ShapeDtypeStruct((N_TOK, RANK), jnp.float32),
        jax.ShapeDtypeStruct((N_TOK, DIM), jnp.float32),
    ),
    compiler_params=pltpu.CompilerParams(use_tc_tiling_on_sc=True),
    scratch_types=[
        pltpu.VMEM((TOK_PER_W,), jnp.int32),
        pltpu.VMEM((2, CKT, RANK), jnp.float32),
        pltpu.VMEM((2, CKT, DIM), jnp.float32),
        pltpu.SemaphoreType.DMA,
        pltpu.SemaphoreType.DMA,
        pltpu.SemaphoreType.DMA,
        pltpu.SemaphoreType.DMA,
    ],
)
def _sc_gathers(
    at_hbm, bw_hbm, idx_hbm, zt_hbm, bg_hbm,
    idx_v, zbuf, bbuf, sem_a, sem_b, sem_c, sem_d,
):
    wid = lax.axis_index("s") * NC + lax.axis_index("c")
    base = wid * TOK_PER_W
    pltpu.sync_copy(idx_hbm.at[pl.ds(base, TOK_PER_W)], idx_v)
    _pipelined_gather(at_hbm, idx_v, zt_hbm, base, zbuf, (sem_a, sem_b), (sem_c, sem_d))
    _pipelined_gather(bw_hbm, idx_v, bg_hbm, base, bbuf, (sem_a, sem_b), (sem_c, sem_d))


TBLK = 2048


def _tc_body(zt_ref, bg_ref, b_ref, out_ref):
    acc = lax.dot_general(
        zt_ref[...], b_ref[...],
        (((1,), (1,)), ((), ())),
        preferred_element_type=jnp.float32,
        precision=lax.Precision.HIGHEST,
    )
    out_ref[...] = bg_ref[...] + acc * SCALING


def _tc_finish(zt, base_g, lora_B):
    return pl.pallas_call(
        _tc_body,
        grid=(N_TOK // TBLK,),
        in_specs=[
            pl.BlockSpec((TBLK, RANK), lambda i: (i, 0)),
            pl.BlockSpec((TBLK, DIM), lambda i: (i, 0)),
            pl.BlockSpec((DIM, RANK), lambda i: (0, 0)),
        ],
        out_specs=pl.BlockSpec((TBLK, DIM), lambda i: (i, 0)),
        out_shape=jax.ShapeDtypeStruct((N_TOK, DIM), jnp.float32),
    )(zt, base_g, lora_B)


def kernel(input_ids, base_weight, lora_A, lora_B):
    n_i, n_j = input_ids.shape
    ids = input_ids.T.reshape(-1).astype(jnp.int32)
    at = lora_A.T
    zt, base_g = _sc_gathers(at, base_weight, ids)
    out = _tc_finish(zt, base_g, lora_B)
    return out.reshape(n_j, n_i, DIM).swapaxes(0, 1)


# speedup vs baseline: 5.2502x; 1.1175x over previous
import functools

import jax
import jax.numpy as jnp
from jax import lax
from jax.experimental import pallas as pl
from jax.experimental.pallas import tpu as pltpu
from jax.experimental.pallas import tpu_sc as plsc

VOCAB = 100000
DIM = 128
RANK = 256
SCALING = 512.0 / 256.0
N_TOK = 1024 * 20

NC = 2
NS = 16
NW = NC * NS
TOK_PER_W = N_TOK // NW
CKT = 128
NCKT = TOK_PER_W // CKT

_mesh = plsc.VectorSubcoreMesh(core_axis_name="c", subcore_axis_name="s")


def _pipelined_gather(table_hbm, idx_v, out_hbm, base, bufs, gsems, ssems):
    g = {}
    s = {}
    g[0] = pltpu.async_copy(
        table_hbm.at[idx_v.at[pl.ds(0, CKT)]], bufs.at[0], gsems[0]
    )
    for c in range(NCKT):
        p = c % 2
        if c + 1 < NCKT:
            if c >= 1:
                s[c - 1].wait()
            g[c + 1] = pltpu.async_copy(
                table_hbm.at[idx_v.at[pl.ds((c + 1) * CKT, CKT)]],
                bufs.at[1 - p],
                gsems[1 - p],
            )
        g[c].wait()
        s[c] = pltpu.async_copy(
            bufs.at[p], out_hbm.at[pl.ds(base + c * CKT, CKT)], ssems[p]
        )
    s[NCKT - 2].wait()
    s[NCKT - 1].wait()


@functools.partial(
    pl.kernel,
    mesh=_mesh,
    out_type=(
        jax.ShapeDtypeStruct((N_TOK, RANK), jnp.float32),
        jax.ShapeDtypeStruct((N_TOK, DIM), jnp.float32),
    ),
    compiler_params=pltpu.CompilerParams(use_tc_tiling_on_sc=True),
    scratch_types=[
        pltpu.VMEM((TOK_PER_W,), jnp.int32),
        pltpu.VMEM((2, CKT, RANK), jnp.float32),
        pltpu.VMEM((2, CKT, DIM), jnp.float32),
        pltpu.SemaphoreType.DMA,
        pltpu.SemaphoreType.DMA,
        pltpu.SemaphoreType.DMA,
        pltpu.SemaphoreType.DMA,
    ],
)
def _sc_gathers(
    at_hbm, bw_hbm, idx_hbm, zt_hbm, bg_hbm,
    idx_v, zbuf, bbuf, sem_a, sem_b, sem_c, sem_d,
):
    wid = lax.axis_index("s") * NC + lax.axis_index("c")
    base = wid * TOK_PER_W
    pltpu.sync_copy(idx_hbm.at[pl.ds(base, TOK_PER_W)], idx_v)
    _pipelined_gather(at_hbm, idx_v, zt_hbm, base, zbuf, (sem_a, sem_b), (sem_c, sem_d))
    _pipelined_gather(bw_hbm, idx_v, bg_hbm, base, bbuf, (sem_a, sem_b), (sem_c, sem_d))


TBLK = 2048


def _tc_body(zt_ref, bg_ref, b_ref, out_ref):
    acc = lax.dot_general(
        zt_ref[...], b_ref[...],
        (((1,), (1,)), ((), ())),
        preferred_element_type=jnp.float32,
    )
    out_ref[...] = bg_ref[...] + acc * SCALING


def _tc_finish(zt, base_g, lora_B):
    return pl.pallas_call(
        _tc_body,
        grid=(N_TOK // TBLK,),
        in_specs=[
            pl.BlockSpec((TBLK, RANK), lambda i: (i, 0)),
            pl.BlockSpec((TBLK, DIM), lambda i: (i, 0)),
            pl.BlockSpec((DIM, RANK), lambda i: (0, 0)),
        ],
        out_specs=pl.BlockSpec((TBLK, DIM), lambda i: (i, 0)),
        out_shape=jax.ShapeDtypeStruct((N_TOK, DIM), jnp.float32),
    )(zt, base_g, lora_B)


def kernel(input_ids, base_weight, lora_A, lora_B):
    n_i, n_j = input_ids.shape
    ids = input_ids.T.reshape(-1).astype(jnp.int32)
    at = lora_A.T
    zt, base_g = _sc_gathers(at, base_weight, ids)
    out = _tc_finish(zt, base_g, lora_B)
    return out.reshape(n_j, n_i, DIM).swapaxes(0, 1)
